# baseline (speedup 1.0000x reference)
import functools

import jax
import jax.numpy as jnp
from jax import lax
from jax.experimental import pallas as pl
from jax.experimental.pallas import tpu as pltpu
from jax.experimental.pallas import tpu_sc as plsc

BATCH = 16384
EMBED_DIM = 32
NUM_ROWS = 1_000_000
NUM_CORES = 2
NUM_SUBCORES = 16
LANES = 16
NUM_WORKERS = NUM_CORES * NUM_SUBCORES
BW = BATCH // NUM_WORKERS
GROUPS = BW // LANES
NW_IDX = BW * EMBED_DIM
IDX_CHUNK = 128
N_CHUNKS = NW_IDX // IDX_CHUNK
B_CHUNKS = BW // IDX_CHUNK
DRAIN_LAG = 32
FLAT = EMBED_DIM * NUM_ROWS

_mesh = plsc.VectorSubcoreMesh(core_axis_name="c", subcore_axis_name="s")


@functools.partial(
    pl.kernel,
    mesh=_mesh,
    out_type=jax.ShapeDtypeStruct((BATCH,), jnp.float32),
    compiler_params=pltpu.CompilerParams(
        needs_layout_passes=False, use_tc_tiling_on_sc=False),
    scratch_types=[
        pltpu.VMEM((BW,), jnp.int32),
        pltpu.VMEM((BW,), jnp.int32),
        pltpu.VMEM((NW_IDX,), jnp.int32),
        pltpu.VMEM((NW_IDX,), jnp.int32),
        pltpu.VMEM((NW_IDX,), jnp.float32),
        pltpu.VMEM((NW_IDX,), jnp.float32),
        pltpu.VMEM((BW,), jnp.float32),
        pltpu.VMEM((BW,), jnp.float32),
        pltpu.VMEM((BW,), jnp.float32),
        pltpu.SemaphoreType.DMA,
        pltpu.SemaphoreType.DMA,
    ],
)
def _mf_sc_kernel(uid_hbm, iid_hbm, uef_hbm, ub_hbm, ief_hbm, ib_hbm, out_hbm,
                  uid_v, iid_v, idx_u, idx_i, pu_v, qi_v, pb_v, qb_v, out_v,
                  sem, bsem):
    wid = lax.axis_index("s") * NUM_CORES + lax.axis_index("c")
    base = wid * BW

    pltpu.sync_copy(uid_hbm.at[pl.ds(base, BW)], uid_v)
    pltpu.sync_copy(iid_hbm.at[pl.ds(base, BW)], iid_v)

    bias_copies = []
    for c in range(B_CHUNKS):
        s = pl.ds(c * IDX_CHUNK, IDX_CHUNK)
        bias_copies.append(pltpu.async_copy(ub_hbm.at[uid_v.at[s]], pb_v.at[s], bsem))
        bias_copies.append(pltpu.async_copy(ib_hbm.at[iid_v.at[s]], qb_v.at[s], bsem))

    def gen_body(g, carry):
        for ids_ref, idx_ref in ((uid_v, idx_u), (iid_v, idx_i)):
            idv = ids_ref[pl.ds(g * LANES, LANES)]
            for d in range(EMBED_DIM):
                idx_ref[pl.ds(d * BW + g * LANES, LANES)] = idv + d * NUM_ROWS
        return carry

    lax.fori_loop(0, GROUPS, gen_body, 0, unroll=False)

    def drain_pair():
        s0 = pl.ds(0, IDX_CHUNK)
        pltpu.make_async_copy(uef_hbm.at[idx_u.at[s0]], pu_v.at[s0], sem).wait()
        pltpu.make_async_copy(ief_hbm.at[idx_i.at[s0]], qi_v.at[s0], sem).wait()

    def dma_body(c, carry):
        s = pl.ds(c * IDX_CHUNK, IDX_CHUNK)
        pltpu.async_copy(uef_hbm.at[idx_u.at[s]], pu_v.at[s], sem)
        pltpu.async_copy(ief_hbm.at[idx_i.at[s]], qi_v.at[s], sem)

        @pl.when(c >= DRAIN_LAG)
        def _():
            drain_pair()

        return carry

    lax.fori_loop(0, N_CHUNKS, dma_body, 0, unroll=False)
    for _ in range(DRAIN_LAG):
        drain_pair()
    for cp in bias_copies:
        cp.wait()

    def compute_body(g, carry):
        s = pl.ds(g * LANES, LANES)
        acc = pb_v[s] + qb_v[s]
        for d in range(EMBED_DIM):
            sd = pl.ds(d * BW + g * LANES, LANES)
            acc = acc + pu_v[sd] * qi_v[sd]
        out_v[s] = acc
        return carry

    lax.fori_loop(0, GROUPS, compute_body, 0, unroll=False)

    pltpu.sync_copy(out_v, out_hbm.at[pl.ds(base, BW)])


def kernel(user_id, item_id, user_embedding, user_bias, item_embedding, item_bias):
    uid = user_id.astype(jnp.int32)
    iid = item_id.astype(jnp.int32)
    return _mf_sc_kernel(uid, iid,
                         user_embedding.T.reshape(-1), user_bias.reshape(-1),
                         item_embedding.T.reshape(-1), item_bias.reshape(-1))


# speedup vs baseline: 5.6673x; 5.6673x over previous
import functools

import jax
import jax.numpy as jnp
from jax import lax
from jax.experimental import pallas as pl
from jax.experimental.pallas import tpu as pltpu
from jax.experimental.pallas import tpu_sc as plsc

BATCH = 16384
EMBED_DIM = 32
PAD_DIM = 128
NUM_ROWS = 1_000_000
NUM_CORES = 2
NUM_SUBCORES = 16
LANES = 16
NUM_WORKERS = NUM_CORES * NUM_SUBCORES
BW = BATCH // NUM_WORKERS
IDX_CHUNK = 128
N_CHUNKS = BW // IDX_CHUNK
CGROUPS = IDX_CHUNK // LANES

_mesh = plsc.VectorSubcoreMesh(core_axis_name="c", subcore_axis_name="s")


@functools.partial(
    pl.kernel,
    mesh=_mesh,
    out_type=jax.ShapeDtypeStruct((BATCH,), jnp.float32),
    compiler_params=pltpu.CompilerParams(
        needs_layout_passes=False, use_tc_tiling_on_sc=True),
    scratch_types=[
        pltpu.VMEM((BW,), jnp.int32),
        pltpu.VMEM((BW,), jnp.int32),
        pltpu.VMEM((2, IDX_CHUNK, PAD_DIM), jnp.float32),
        pltpu.VMEM((2, IDX_CHUNK, PAD_DIM), jnp.float32),
        pltpu.VMEM((BW,), jnp.float32),
        pltpu.VMEM((BW,), jnp.float32),
        pltpu.VMEM((BW,), jnp.float32),
        pltpu.SemaphoreType.DMA,
        pltpu.SemaphoreType.DMA,
    ],
)
def _mf_sc_kernel(uid_hbm, iid_hbm, uep_hbm, ub_hbm, iep_hbm, ib_hbm, out_hbm,
                  uid_v, iid_v, urows, irows, pb_v, qb_v, out_v, sem, bsem):
    wid = lax.axis_index("s") * NUM_CORES + lax.axis_index("c")
    base = wid * BW

    pltpu.sync_copy(uid_hbm.at[pl.ds(base, BW)], uid_v)
    pltpu.sync_copy(iid_hbm.at[pl.ds(base, BW)], iid_v)

    bias_copies = []
    for c in range(N_CHUNKS):
        s = pl.ds(c * IDX_CHUNK, IDX_CHUNK)
        bias_copies.append(pltpu.async_copy(ub_hbm.at[uid_v.at[s]], pb_v.at[s], bsem))
        bias_copies.append(pltpu.async_copy(ib_hbm.at[iid_v.at[s]], qb_v.at[s], bsem))

    def fire_chunk(c, buf):
        s = pl.ds(c * IDX_CHUNK, IDX_CHUNK)
        pltpu.async_copy(uep_hbm.at[uid_v.at[s]], urows.at[buf], sem)
        pltpu.async_copy(iep_hbm.at[iid_v.at[s]], irows.at[buf], sem)

    def drain_chunk(buf):
        pltpu.make_async_copy(uep_hbm.at[uid_v.at[pl.ds(0, IDX_CHUNK)]],
                              urows.at[buf], sem).wait()
        pltpu.make_async_copy(iep_hbm.at[iid_v.at[pl.ds(0, IDX_CHUNK)]],
                              irows.at[buf], sem).wait()

    lanes = lax.iota(jnp.int32, LANES)

    def compute_chunk(c, buf):
        def group_body(g, carry):
            rows = g * LANES + lanes
            s = pl.ds(c * IDX_CHUNK + g * LANES, LANES)
            acc = pb_v[s] + qb_v[s]
            for d in range(EMBED_DIM):
                dcol = jnp.full((LANES,), d, jnp.int32)
                acc = acc + (plsc.load_gather(urows.at[buf], [rows, dcol])
                             * plsc.load_gather(irows.at[buf], [rows, dcol]))
            out_v[s] = acc
            return carry

        lax.fori_loop(0, CGROUPS, group_body, 0, unroll=False)

    fire_chunk(0, 0)
    for c in range(N_CHUNKS):
        if c + 1 < N_CHUNKS:
            fire_chunk(c + 1, (c + 1) % 2)
        drain_chunk(c % 2)
        if c == 0:
            for cp in bias_copies:
                cp.wait()
        compute_chunk(c, c % 2)

    pltpu.sync_copy(out_v, out_hbm.at[pl.ds(base, BW)])


def kernel(user_id, item_id, user_embedding, user_bias, item_embedding, item_bias):
    uid = user_id.astype(jnp.int32)
    iid = item_id.astype(jnp.int32)
    uep = jnp.pad(user_embedding, ((0, 0), (0, PAD_DIM - EMBED_DIM)))
    iep = jnp.pad(item_embedding, ((0, 0), (0, PAD_DIM - EMBED_DIM)))
    return _mf_sc_kernel(uid, iid, uep, user_bias.reshape(-1),
                         iep, item_bias.reshape(-1))
